# Initial kernel scaffold; baseline (speedup 1.0000x reference)
import jax
import jax.numpy as jnp
from jax.experimental import pallas as pl


def kernel(x, edge_index, edge_attr, Wf0, bf0, Ws0, bs0, Wu0, bu0, Wf1, bf1, Ws1, bs1, Wu1, bu1, Wf2, bf2, Ws2, bs2, Wu2, bu2):
    raise NotImplementedError("write your pallas kernel here")


# speedup vs baseline: 3.4147x; 3.4147x over previous
import functools
import jax
import jax.numpy as jnp
from jax import lax
from jax.experimental import pallas as pl
from jax.experimental.pallas import tpu as pltpu
from jax.experimental.pallas import tpu_sc as plsc

N = 10000
E = 640000
D = 128
DE = 16
DZ = 256
NC = 2
NS = 16
NW = NC * NS
EPW = E // NW
CH = 80
NCHUNK = EPW // CH
RPT = 624
RTAIL = N - NS * RPT

_f32 = jnp.float32


_BR = 1000
_BE = 2000


def _node0_body(x_ref, wp_ref, wq_ref, wu_ref, bu_ref, p_ref, q_ref, u_ref):
    h = x_ref[...]
    p_ref[...] = jnp.dot(h, wp_ref[...], preferred_element_type=_f32)
    q_ref[...] = jnp.dot(h, wq_ref[...], preferred_element_type=_f32)
    u_ref[...] = jnp.dot(h, wu_ref[...], preferred_element_type=_f32) + bu_ref[...]


def _node12_body(a0_ref, a1_ref, up_ref, wp_ref, wq_ref, wu_ref, bu_ref,
                 p_ref, q_ref, u_ref):
    h = a0_ref[...] + a1_ref[...] + up_ref[...]
    p_ref[...] = jnp.dot(h, wp_ref[...], preferred_element_type=_f32)
    q_ref[...] = jnp.dot(h, wq_ref[...], preferred_element_type=_f32)
    u_ref[...] = jnp.dot(h, wu_ref[...], preferred_element_type=_f32) + bu_ref[...]


def _edge_body(gd_ref, gs_ref, ea_ref, we_ref, be_ref, m_ref):
    t = (gd_ref[...] + gs_ref[...] + be_ref[...]
         + jnp.dot(ea_ref[...], we_ref[...], preferred_element_type=_f32))
    m_ref[...] = jax.nn.sigmoid(t[:, :D]) * jax.nn.softplus(t[:, D:])


def _final_body(a0_ref, a1_ref, up_ref, o_ref):
    o_ref[...] = a0_ref[...] + a1_ref[...] + up_ref[...]


def _node_call(first):
    body = _node0_body if first else _node12_body
    grid = N // _BR
    hspec = [pl.BlockSpec((_BR, D), lambda i: (i, 0))]
    if not first:
        hspec = [pl.BlockSpec((_BR, D), lambda i: (i, 0)),
                 pl.BlockSpec((_BR, D), lambda i: (i + grid, 0)),
                 pl.BlockSpec((_BR, D), lambda i: (i, 0))]
    wspecs = [pl.BlockSpec((D, DZ), lambda i: (0, 0)),
              pl.BlockSpec((D, DZ), lambda i: (0, 0)),
              pl.BlockSpec((D, D), lambda i: (0, 0)),
              pl.BlockSpec((1, D), lambda i: (0, 0))]
    return pl.pallas_call(
        body,
        grid=(grid,),
        in_specs=hspec + wspecs,
        out_specs=[pl.BlockSpec((_BR, DZ), lambda i: (i, 0)),
                   pl.BlockSpec((_BR, DZ), lambda i: (i, 0)),
                   pl.BlockSpec((_BR, D), lambda i: (i, 0))],
        out_shape=[jax.ShapeDtypeStruct((N, DZ), _f32),
                   jax.ShapeDtypeStruct((N, DZ), _f32),
                   jax.ShapeDtypeStruct((N, D), _f32)],
    )


_edge_call = pl.pallas_call(
    _edge_body,
    grid=(E // _BE,),
    in_specs=[pl.BlockSpec((_BE, DZ), lambda i: (i, 0)),
              pl.BlockSpec((_BE, DZ), lambda i: (i, 0)),
              pl.BlockSpec((_BE, DE), lambda i: (i, 0)),
              pl.BlockSpec((DE, DZ), lambda i: (0, 0)),
              pl.BlockSpec((1, DZ), lambda i: (0, 0))],
    out_specs=pl.BlockSpec((_BE, D), lambda i: (i, 0)),
    out_shape=jax.ShapeDtypeStruct((E, D), _f32),
)

_final_call = pl.pallas_call(
    _final_body,
    grid=(N // _BR,),
    in_specs=[pl.BlockSpec((_BR, D), lambda i: (i, 0)),
              pl.BlockSpec((_BR, D), lambda i: (i + N // _BR, 0)),
              pl.BlockSpec((_BR, D), lambda i: (i, 0))],
    out_specs=pl.BlockSpec((_BR, D), lambda i: (i, 0)),
    out_shape=jax.ShapeDtypeStruct((N, D), _f32),
)


@functools.lru_cache(maxsize=None)
def _sc_kernels():
    mesh = plsc.VectorSubcoreMesh(core_axis_name="c", subcore_axis_name="s",
                                  num_cores=NC, num_subcores=NS)

    @functools.partial(
        pl.kernel,
        out_type=[jax.ShapeDtypeStruct((E, DZ), _f32),
                  jax.ShapeDtypeStruct((E, DZ), _f32)],
        mesh=mesh,
        scratch_types=[pltpu.VMEM((NCHUNK, CH), jnp.int32),
                       pltpu.VMEM((NCHUNK, CH), jnp.int32),
                       pltpu.VMEM((CH, DZ), _f32),
                       pltpu.VMEM((CH, DZ), _f32),
                       pltpu.SemaphoreType.DMA,
                       pltpu.SemaphoreType.DMA],
    )
    def sc_gather(p_hbm, q_hbm, dst_hbm, src_hbm, gd_hbm, gs_hbm,
                  di_v, si_v, bd_v, bq_v, semd, semq):
        wid = lax.axis_index("s") * NC + lax.axis_index("c")
        pltpu.sync_copy(dst_hbm.at[wid], di_v)
        pltpu.sync_copy(src_hbm.at[wid], si_v)

        def chunk(j, carry):
            cpd = pltpu.async_copy(p_hbm.at[di_v.at[j]], bd_v, semd)
            cpq = pltpu.async_copy(q_hbm.at[si_v.at[j]], bq_v, semq)
            cpd.wait()
            cpq.wait()
            off = wid * EPW + j * CH
            pltpu.sync_copy(bd_v, gd_hbm.at[pl.ds(off, CH)])
            pltpu.sync_copy(bq_v, gs_hbm.at[pl.ds(off, CH)])
            return carry

        lax.fori_loop(0, NCHUNK, chunk, 0)

    @functools.partial(
        pl.kernel,
        out_type=jax.ShapeDtypeStruct((2 * N, D), _f32),
        mesh=mesh,
        scratch_types=[pltpu.VMEM((NCHUNK, CH), jnp.int32),
                       pltpu.VMEM((CH, D), _f32),
                       pltpu.VMEM_SHARED((N, D), _f32)],
    )
    def sc_scatter(m_hbm, dst_hbm, zero_hbm, agg_hbm, di_v, mv, agg_sh):
        cid = lax.axis_index("c")
        sid = lax.axis_index("s")
        wid = sid * NC + cid
        r0 = sid * RPT
        pltpu.sync_copy(zero_hbm.at[pl.ds(r0, RPT)], agg_sh.at[pl.ds(r0, RPT)])

        @pl.when(sid == NS - 1)
        def _():
            pltpu.sync_copy(zero_hbm.at[pl.ds(NS * RPT, RTAIL)],
                            agg_sh.at[pl.ds(NS * RPT, RTAIL)])

        pltpu.sync_copy(dst_hbm.at[wid], di_v)
        plsc.subcore_barrier()

        def chunk(j, carry):
            off = wid * EPW + j * CH
            pltpu.sync_copy(m_hbm.at[pl.ds(off, CH)], mv)
            pltpu.sync_copy(mv, agg_sh.at[di_v.at[j]], add=True)
            return carry

        lax.fori_loop(0, NCHUNK, chunk, 0)
        plsc.subcore_barrier()
        pltpu.sync_copy(agg_sh.at[pl.ds(r0, RPT)],
                        agg_hbm.at[pl.ds(cid * N + r0, RPT)])

        @pl.when(sid == NS - 1)
        def _():
            pltpu.sync_copy(agg_sh.at[pl.ds(NS * RPT, RTAIL)],
                            agg_hbm.at[pl.ds(cid * N + NS * RPT, RTAIL)])

    return sc_gather, sc_scatter


def kernel(x, edge_index, edge_attr, Wf0, bf0, Ws0, bs0, Wu0, bu0,
           Wf1, bf1, Ws1, bs1, Wu1, bu1, Wf2, bf2, Ws2, bs2, Wu2, bu2):
    sc_gather, sc_scatter = _sc_kernels()
    src = edge_index[0].reshape(NW, NCHUNK, CH)
    dst = edge_index[1].reshape(NW, NCHUNK, CH)
    zero = jnp.zeros((N, D), _f32)

    layers = [(Wf0, bf0, Ws0, bs0, Wu0, bu0),
              (Wf1, bf1, Ws1, bs1, Wu1, bu1),
              (Wf2, bf2, Ws2, bs2, Wu2, bu2)]

    agg = None
    u_prev = None
    for li, (Wf, bf, Ws, bs, Wu, bu) in enumerate(layers):
        wp = jnp.concatenate([Wf[:D], Ws[:D]], axis=1)
        wq = jnp.concatenate([Wf[D:2 * D], Ws[D:2 * D]], axis=1)
        we = jnp.concatenate([Wf[2 * D:], Ws[2 * D:]], axis=1)
        be = jnp.concatenate([bf, bs]).reshape(1, DZ)
        bu2d = bu.reshape(1, D)
        if li == 0:
            p, q, u = _node_call(True)(x, wp, wq, Wu, bu2d)
        else:
            p, q, u = _node_call(False)(agg, agg, u_prev, wp, wq, Wu, bu2d)
        gd, gs = sc_gather(p, q, dst, src)
        m = _edge_call(gd, gs, edge_attr, we, be)
        agg = sc_scatter(m, dst, zero)
        u_prev = u

    out = _final_call(agg, agg, u_prev)
    return out.reshape(1, N, D)


# speedup vs baseline: 6.2175x; 1.8208x over previous
import functools
import jax
import jax.numpy as jnp
from jax import lax
from jax.experimental import pallas as pl
from jax.experimental.pallas import tpu as pltpu
from jax.experimental.pallas import tpu_sc as plsc

N = 10000
E = 640000
D = 128
DE = 16
NC = 2
NS = 16
NW = NC * NS
EPW = E // NW
CH = 80
NCHUNK = EPW // CH
RPT = 624
RTAIL = N - NS * RPT

_f32 = jnp.float32
_bf16 = jnp.bfloat16
_i32 = jnp.int32
_u32 = jnp.uint32


def _pack(f_arm, s_arm):
    fb = lax.bitcast_convert_type(f_arm.astype(_bf16), jnp.uint16).astype(_u32)
    sb = lax.bitcast_convert_type(s_arm.astype(_bf16), jnp.uint16).astype(_u32)
    return lax.bitcast_convert_type(fb | (sb << 16), _i32)


_BR = 1000
_BE = 2000


def _node0_body(x_ref, wfi, wsi, wfj, wsj, wu, bu, p_ref, q_ref, u_ref):
    h = x_ref[...]
    p_ref[...] = _pack(jnp.dot(h, wfi[...], preferred_element_type=_f32),
                       jnp.dot(h, wsi[...], preferred_element_type=_f32))
    q_ref[...] = _pack(jnp.dot(h, wfj[...], preferred_element_type=_f32),
                       jnp.dot(h, wsj[...], preferred_element_type=_f32))
    u_ref[...] = jnp.dot(h, wu[...], preferred_element_type=_f32) + bu[...]


def _node12_body(a0_ref, a1_ref, up_ref, wfi, wsi, wfj, wsj, wu, bu,
                 p_ref, q_ref, u_ref):
    h = a0_ref[...] + a1_ref[...] + up_ref[...]
    p_ref[...] = _pack(jnp.dot(h, wfi[...], preferred_element_type=_f32),
                       jnp.dot(h, wsi[...], preferred_element_type=_f32))
    q_ref[...] = _pack(jnp.dot(h, wfj[...], preferred_element_type=_f32),
                       jnp.dot(h, wsj[...], preferred_element_type=_f32))
    u_ref[...] = jnp.dot(h, wu[...], preferred_element_type=_f32) + bu[...]


def _edge_body(fs_ref, ea_ref, wfe, wse, bf_ref, bs_ref, m_ref):
    fs = lax.bitcast_convert_type(fs_ref[...], _u32)
    f_arm = lax.bitcast_convert_type(fs << 16, _f32)
    s_arm = lax.bitcast_convert_type(fs & _u32(0xFFFF0000), _f32)
    ea = ea_ref[...]
    f = f_arm + jnp.dot(ea, wfe[...], preferred_element_type=_f32) + bf_ref[...]
    s = s_arm + jnp.dot(ea, wse[...], preferred_element_type=_f32) + bs_ref[...]
    m_ref[...] = jax.nn.sigmoid(f) * jax.nn.softplus(s)


def _final_body(a0_ref, a1_ref, up_ref, o_ref):
    o_ref[...] = a0_ref[...] + a1_ref[...] + up_ref[...]


def _node_call(first):
    body = _node0_body if first else _node12_body
    grid = N // _BR
    hspec = [pl.BlockSpec((_BR, D), lambda i: (i, 0))]
    if not first:
        hspec = [pl.BlockSpec((_BR, D), lambda i: (i, 0)),
                 pl.BlockSpec((_BR, D), lambda i: (i + grid, 0)),
                 pl.BlockSpec((_BR, D), lambda i: (i, 0))]
    wspecs = [pl.BlockSpec((D, D), lambda i: (0, 0))] * 5 + \
             [pl.BlockSpec((1, D), lambda i: (0, 0))]
    return pl.pallas_call(
        body,
        grid=(grid,),
        in_specs=hspec + wspecs,
        out_specs=[pl.BlockSpec((_BR, D), lambda i: (i, 0)),
                   pl.BlockSpec((_BR, D), lambda i: (i, 0)),
                   pl.BlockSpec((_BR, D), lambda i: (i, 0))],
        out_shape=[jax.ShapeDtypeStruct((N, D), _i32),
                   jax.ShapeDtypeStruct((N, D), _i32),
                   jax.ShapeDtypeStruct((N, D), _f32)],
    )


_edge_call = pl.pallas_call(
    _edge_body,
    grid=(E // _BE,),
    in_specs=[pl.BlockSpec((_BE, D), lambda i: (i, 0)),
              pl.BlockSpec((_BE, DE), lambda i: (i, 0)),
              pl.BlockSpec((DE, D), lambda i: (0, 0)),
              pl.BlockSpec((DE, D), lambda i: (0, 0)),
              pl.BlockSpec((1, D), lambda i: (0, 0)),
              pl.BlockSpec((1, D), lambda i: (0, 0))],
    out_specs=pl.BlockSpec((_BE, D), lambda i: (i, 0)),
    out_shape=jax.ShapeDtypeStruct((E, D), _f32),
)

_final_call = pl.pallas_call(
    _final_body,
    grid=(N // _BR,),
    in_specs=[pl.BlockSpec((_BR, D), lambda i: (i, 0)),
              pl.BlockSpec((_BR, D), lambda i: (i + N // _BR, 0)),
              pl.BlockSpec((_BR, D), lambda i: (i, 0))],
    out_specs=pl.BlockSpec((_BR, D), lambda i: (i, 0)),
    out_shape=jax.ShapeDtypeStruct((N, D), _f32),
)


@functools.lru_cache(maxsize=None)
def _sc_kernels():
    mesh = plsc.VectorSubcoreMesh(core_axis_name="c", subcore_axis_name="s",
                                  num_cores=NC, num_subcores=NS)

    @functools.partial(
        pl.kernel,
        out_type=jax.ShapeDtypeStruct((E, D), _i32),
        mesh=mesh,
        scratch_types=[pltpu.VMEM((NCHUNK, CH), _i32),
                       pltpu.VMEM((NCHUNK, CH), _i32),
                       pltpu.VMEM((2, CH, D), _i32),
                       pltpu.VMEM((2, CH, D), _i32),
                       pltpu.VMEM((2, CH, D), _i32),
                       pltpu.SemaphoreType.DMA,
                       pltpu.SemaphoreType.DMA,
                       pltpu.SemaphoreType.DMA,
                       pltpu.SemaphoreType.DMA,
                       pltpu.SemaphoreType.DMA,
                       pltpu.SemaphoreType.DMA],
    )
    def sc_gather(p_hbm, q_hbm, dst_hbm, src_hbm, fs_hbm,
                  di_v, si_v, bd_v, bq_v, wb_v,
                  semp0, semp1, semq0, semq1, semw0, semw1):
        semp = (semp0, semp1)
        semq = (semq0, semq1)
        semw = (semw0, semw1)
        wid = lax.axis_index("s") * NC + lax.axis_index("c")
        base = wid * EPW
        pltpu.sync_copy(dst_hbm.at[wid], di_v)
        pltpu.sync_copy(src_hbm.at[wid], si_v)

        def issue(jj, b):
            pltpu.async_copy(p_hbm.at[di_v.at[jj]], bd_v.at[b], semp[b])
            pltpu.async_copy(q_hbm.at[si_v.at[jj]], bq_v.at[b], semq[b])

        issue(0, 0)
        issue(1, 1)

        @pl.loop(0, NCHUNK, step=2)
        def _(j):
            for b in range(2):
                jj = j + b
                pltpu.make_async_copy(p_hbm.at[pl.ds(0, CH)], bd_v.at[b],
                                      semp[b]).wait()
                pltpu.make_async_copy(q_hbm.at[pl.ds(0, CH)], bq_v.at[b],
                                      semq[b]).wait()

                @pl.when(jj >= 2)
                def _():
                    pltpu.make_async_copy(
                        wb_v.at[b], fs_hbm.at[pl.ds(0, CH)], semw[b]).wait()

                @pl.loop(0, CH)
                def _(r):
                    for g in range(D // 16):
                        sl = pl.ds(g * 16, 16)
                        xd = bd_v[b, r, sl]
                        xq = bq_v[b, r, sl]
                        bc = lax.bitcast_convert_type
                        fsum = bc(xd << 16, _f32) + bc(xq << 16, _f32)
                        ssum = (bc(xd & _i32(-65536), _f32)
                                + bc(xq & _i32(-65536), _f32))
                        rb = _i32(0x8000)
                        lo = ((bc(fsum, _i32) + rb) >> 16) & _i32(0xFFFF)
                        hi = (bc(ssum, _i32) + rb) & _i32(-65536)
                        wb_v[b, r, sl] = lo | hi

                pltpu.async_copy(wb_v.at[b],
                                 fs_hbm.at[pl.ds(base + jj * CH, CH)], semw[b])

                @pl.when(jj + 2 < NCHUNK)
                def _():
                    issue(jj + 2, b)

        for b in range(2):
            pltpu.make_async_copy(wb_v.at[b], fs_hbm.at[pl.ds(0, CH)],
                                  semw[b]).wait()

    @functools.partial(
        pl.kernel,
        out_type=jax.ShapeDtypeStruct((2 * N, D), _f32),
        mesh=mesh,
        scratch_types=[pltpu.VMEM((2, CH), _i32),
                       pltpu.VMEM((2, CH, D), _f32),
                       pltpu.VMEM_SHARED((N, D), _f32),
                       pltpu.SemaphoreType.DMA,
                       pltpu.SemaphoreType.DMA,
                       pltpu.SemaphoreType.DMA,
                       pltpu.SemaphoreType.DMA],
    )
    def sc_scatter(m_hbm, dst_hbm, zero_hbm, agg_hbm, di_v, mv, agg_sh,
                   semm0, semm1, semi0, semi1):
        semm = (semm0, semm1)
        semi = (semi0, semi1)
        cid = lax.axis_index("c")
        sid = lax.axis_index("s")
        wid = sid * NC + cid
        base = wid * EPW
        r0 = sid * RPT
        for b in range(2):
            pltpu.async_copy(dst_hbm.at[pl.ds(base + b * CH, CH)],
                             di_v.at[b], semi[b])
            pltpu.async_copy(m_hbm.at[pl.ds(base + b * CH, CH)],
                             mv.at[b], semm[b])
        pltpu.sync_copy(zero_hbm.at[pl.ds(r0, RPT)], agg_sh.at[pl.ds(r0, RPT)])

        @pl.when(sid == NS - 1)
        def _():
            pltpu.sync_copy(zero_hbm.at[pl.ds(NS * RPT, RTAIL)],
                            agg_sh.at[pl.ds(NS * RPT, RTAIL)])

        plsc.subcore_barrier()

        @pl.loop(0, NCHUNK, step=2)
        def _(j):
            for b in range(2):
                jj = j + b
                pltpu.make_async_copy(dst_hbm.at[pl.ds(0, CH)], di_v.at[b],
                                      semi[b]).wait()
                pltpu.make_async_copy(m_hbm.at[pl.ds(0, CH)], mv.at[b],
                                      semm[b]).wait()
                pltpu.sync_copy(mv.at[b], agg_sh.at[di_v.at[b]], add=True)

                @pl.when(jj + 2 < NCHUNK)
                def _():
                    pltpu.async_copy(
                        dst_hbm.at[pl.ds(base + (jj + 2) * CH, CH)],
                        di_v.at[b], semi[b])
                    pltpu.async_copy(m_hbm.at[pl.ds(base + (jj + 2) * CH, CH)],
                                     mv.at[b], semm[b])

        plsc.subcore_barrier()
        pltpu.sync_copy(agg_sh.at[pl.ds(r0, RPT)],
                        agg_hbm.at[pl.ds(cid * N + r0, RPT)])

        @pl.when(sid == NS - 1)
        def _():
            pltpu.sync_copy(agg_sh.at[pl.ds(NS * RPT, RTAIL)],
                            agg_hbm.at[pl.ds(cid * N + NS * RPT, RTAIL)])

    return sc_gather, sc_scatter


def kernel(x, edge_index, edge_attr, Wf0, bf0, Ws0, bs0, Wu0, bu0,
           Wf1, bf1, Ws1, bs1, Wu1, bu1, Wf2, bf2, Ws2, bs2, Wu2, bu2):
    sc_gather, sc_scatter = _sc_kernels()
    src = edge_index[0].reshape(NW, NCHUNK, CH)
    dst = edge_index[1].reshape(NW, NCHUNK, CH)
    dst_flat = edge_index[1]
    zero = jnp.zeros((N, D), _f32)

    layers = [(Wf0, bf0, Ws0, bs0, Wu0, bu0),
              (Wf1, bf1, Ws1, bs1, Wu1, bu1),
              (Wf2, bf2, Ws2, bs2, Wu2, bu2)]

    agg = None
    u_prev = None
    for li, (Wf, bf, Ws, bs, Wu, bu) in enumerate(layers):
        wfi, wfj, wfe = Wf[:D], Wf[D:2 * D], Wf[2 * D:]
        wsi, wsj, wse = Ws[:D], Ws[D:2 * D], Ws[2 * D:]
        bf2d = bf.reshape(1, D)
        bs2d = bs.reshape(1, D)
        bu2d = bu.reshape(1, D)
        if li == 0:
            p, q, u = _node_call(True)(x, wfi, wsi, wfj, wsj, Wu, bu2d)
        else:
            p, q, u = _node_call(False)(agg, agg, u_prev,
                                        wfi, wsi, wfj, wsj, Wu, bu2d)
        fs = sc_gather(p, q, dst, src)
        m = _edge_call(fs, edge_attr, wfe, wse, bf2d, bs2d)
        agg = sc_scatter(m, dst_flat, zero)
        u_prev = u

    out = _final_call(agg, agg, u_prev)
    return out.reshape(1, N, D)


# speedup vs baseline: 6.7898x; 1.0921x over previous
import functools
import jax
import jax.numpy as jnp
from jax import lax
from jax.experimental import pallas as pl
from jax.experimental.pallas import tpu as pltpu
from jax.experimental.pallas import tpu_sc as plsc

N = 10000
E = 640000
EH = E // 2
D = 128
DE = 16
NC = 2
NS = 16
NW = NC * NS
EPW = EH // NW
CH = 40
NCHUNK = EPW // CH
RPT = 624
RTAIL = N - NS * RPT

_f32 = jnp.float32
_i32 = jnp.int32
_u32 = jnp.uint32


def _pack(f_arm, s_arm):
    fb = lax.bitcast_convert_type(f_arm.astype(jnp.bfloat16), jnp.uint16).astype(_u32)
    sb = lax.bitcast_convert_type(s_arm.astype(jnp.bfloat16), jnp.uint16).astype(_u32)
    return lax.bitcast_convert_type(fb | (sb << 16), _i32)


_BR = 1000
_BE = 2000


def _node0_body(x_ref, wfi, wsi, wfj, wsj, wu, bu, p_ref, q_ref, u_ref):
    h = x_ref[...]
    p_ref[...] = _pack(jnp.dot(h, wfi[...], preferred_element_type=_f32),
                       jnp.dot(h, wsi[...], preferred_element_type=_f32))
    q_ref[...] = _pack(jnp.dot(h, wfj[...], preferred_element_type=_f32),
                       jnp.dot(h, wsj[...], preferred_element_type=_f32))
    u_ref[...] = jnp.dot(h, wu[...], preferred_element_type=_f32) + bu[...]


def _node12_body(a0_ref, a1_ref, a2_ref, a3_ref, up_ref,
                 wfi, wsi, wfj, wsj, wu, bu, p_ref, q_ref, u_ref):
    h = (a0_ref[...] + a1_ref[...]) + (a2_ref[...] + a3_ref[...]) + up_ref[...]
    p_ref[...] = _pack(jnp.dot(h, wfi[...], preferred_element_type=_f32),
                       jnp.dot(h, wsi[...], preferred_element_type=_f32))
    q_ref[...] = _pack(jnp.dot(h, wfj[...], preferred_element_type=_f32),
                       jnp.dot(h, wsj[...], preferred_element_type=_f32))
    u_ref[...] = jnp.dot(h, wu[...], preferred_element_type=_f32) + bu[...]


def _edge_body(fs_ref, ea_ref, wfe, wse, bf_ref, bs_ref, m_ref):
    fs = lax.bitcast_convert_type(fs_ref[...], _u32)
    f_arm = lax.bitcast_convert_type(fs << 16, _f32)
    s_arm = lax.bitcast_convert_type(fs & _u32(0xFFFF0000), _f32)
    ea = ea_ref[...]
    f = f_arm + jnp.dot(ea, wfe[...], preferred_element_type=_f32) + bf_ref[...]
    s = s_arm + jnp.dot(ea, wse[...], preferred_element_type=_f32) + bs_ref[...]
    sig = 1.0 / (1.0 + jnp.exp(-f))
    sp = jnp.maximum(s, 0.0) + jnp.log1p(jnp.exp(-jnp.abs(s)))
    m_ref[...] = sig * sp


def _final_body(a0_ref, a1_ref, a2_ref, a3_ref, up_ref, o_ref):
    o_ref[...] = (a0_ref[...] + a1_ref[...]) + (a2_ref[...] + a3_ref[...]) \
        + up_ref[...]


def _node_call(first):
    body = _node0_body if first else _node12_body
    grid = N // _BR
    if first:
        hspec = [pl.BlockSpec((_BR, D), lambda i: (i, 0))]
    else:
        hspec = [pl.BlockSpec((_BR, D), lambda i: (i, 0)),
                 pl.BlockSpec((_BR, D), lambda i: (i + grid, 0)),
                 pl.BlockSpec((_BR, D), lambda i: (i, 0)),
                 pl.BlockSpec((_BR, D), lambda i: (i + grid, 0)),
                 pl.BlockSpec((_BR, D), lambda i: (i, 0))]
    wspecs = [pl.BlockSpec((D, D), lambda i: (0, 0))] * 5 + \
             [pl.BlockSpec((1, D), lambda i: (0, 0))]
    return pl.pallas_call(
        body,
        grid=(grid,),
        in_specs=hspec + wspecs,
        out_specs=[pl.BlockSpec((_BR, D), lambda i: (i, 0)),
                   pl.BlockSpec((_BR, D), lambda i: (i, 0)),
                   pl.BlockSpec((_BR, D), lambda i: (i, 0))],
        out_shape=[jax.ShapeDtypeStruct((N, D), _i32),
                   jax.ShapeDtypeStruct((N, D), _i32),
                   jax.ShapeDtypeStruct((N, D), _f32)],
    )


_edge_call = pl.pallas_call(
    _edge_body,
    grid=(EH // _BE,),
    in_specs=[pl.BlockSpec((_BE, D), lambda i: (i, 0)),
              pl.BlockSpec((_BE, DE), lambda i: (i, 0)),
              pl.BlockSpec((DE, D), lambda i: (0, 0)),
              pl.BlockSpec((DE, D), lambda i: (0, 0)),
              pl.BlockSpec((1, D), lambda i: (0, 0)),
              pl.BlockSpec((1, D), lambda i: (0, 0))],
    out_specs=pl.BlockSpec((_BE, D), lambda i: (i, 0)),
    out_shape=jax.ShapeDtypeStruct((EH, D), _f32),
)

_final_call = pl.pallas_call(
    _final_body,
    grid=(N // _BR,),
    in_specs=[pl.BlockSpec((_BR, D), lambda i: (i, 0)),
              pl.BlockSpec((_BR, D), lambda i: (i + N // _BR, 0)),
              pl.BlockSpec((_BR, D), lambda i: (i, 0)),
              pl.BlockSpec((_BR, D), lambda i: (i + N // _BR, 0)),
              pl.BlockSpec((_BR, D), lambda i: (i, 0))],
    out_specs=pl.BlockSpec((_BR, D), lambda i: (i, 0)),
    out_shape=jax.ShapeDtypeStruct((N, D), _f32),
)


@functools.lru_cache(maxsize=None)
def _sc_kernels():
    mesh = plsc.VectorSubcoreMesh(core_axis_name="c", subcore_axis_name="s",
                                  num_cores=NC, num_subcores=NS)

    @functools.partial(
        pl.kernel,
        out_type=jax.ShapeDtypeStruct((EH, D), _i32),
        mesh=mesh,
        scratch_types=[pltpu.VMEM((NCHUNK, CH), _i32),
                       pltpu.VMEM((NCHUNK, CH), _i32),
                       pltpu.VMEM((2, CH, D), _i32),
                       pltpu.VMEM((2, CH, D), _i32),
                       pltpu.VMEM((2, CH, D), _i32),
                       pltpu.SemaphoreType.DMA,
                       pltpu.SemaphoreType.DMA,
                       pltpu.SemaphoreType.DMA,
                       pltpu.SemaphoreType.DMA,
                       pltpu.SemaphoreType.DMA,
                       pltpu.SemaphoreType.DMA],
    )
    def sc_gather(p_hbm, q_hbm, dst_hbm, src_hbm, fs_hbm,
                  di_v, si_v, bd_v, bq_v, wb_v,
                  semp0, semp1, semq0, semq1, semw0, semw1):
        semp = (semp0, semp1)
        semq = (semq0, semq1)
        semw = (semw0, semw1)
        wid = lax.axis_index("s") * NC + lax.axis_index("c")
        base = wid * EPW
        pltpu.sync_copy(dst_hbm.at[wid], di_v)
        pltpu.sync_copy(src_hbm.at[wid], si_v)

        def issue(jj, b):
            pltpu.async_copy(p_hbm.at[di_v.at[jj]], bd_v.at[b], semp[b])
            pltpu.async_copy(q_hbm.at[si_v.at[jj]], bq_v.at[b], semq[b])

        issue(0, 0)
        issue(1, 1)

        @pl.loop(0, NCHUNK, step=2)
        def _(j):
            for b in range(2):
                jj = j + b
                pltpu.make_async_copy(p_hbm.at[pl.ds(0, CH)], bd_v.at[b],
                                      semp[b]).wait()
                pltpu.make_async_copy(q_hbm.at[pl.ds(0, CH)], bq_v.at[b],
                                      semq[b]).wait()

                @pl.when(jj >= 2)
                def _():
                    pltpu.make_async_copy(
                        wb_v.at[b], fs_hbm.at[pl.ds(0, CH)], semw[b]).wait()

                @pl.loop(0, CH)
                def _(r):
                    for g in range(D // 16):
                        sl = pl.ds(g * 16, 16)
                        xd = bd_v[b, r, sl]
                        xq = bq_v[b, r, sl]
                        bc = lax.bitcast_convert_type
                        fsum = bc(xd << 16, _f32) + bc(xq << 16, _f32)
                        ssum = (bc(xd & _i32(-65536), _f32)
                                + bc(xq & _i32(-65536), _f32))
                        rb = _i32(0x8000)
                        lo = ((bc(fsum, _i32) + rb) >> 16) & _i32(0xFFFF)
                        hi = (bc(ssum, _i32) + rb) & _i32(-65536)
                        wb_v[b, r, sl] = lo | hi

                pltpu.async_copy(wb_v.at[b],
                                 fs_hbm.at[pl.ds(base + jj * CH, CH)], semw[b])

                @pl.when(jj + 2 < NCHUNK)
                def _():
                    issue(jj + 2, b)

        for b in range(2):
            pltpu.make_async_copy(wb_v.at[b], fs_hbm.at[pl.ds(0, CH)],
                                  semw[b]).wait()

    @functools.partial(
        pl.kernel,
        out_type=jax.ShapeDtypeStruct((2 * N, D), _f32),
        mesh=mesh,
        scratch_types=[pltpu.VMEM((2, CH), _i32),
                       pltpu.VMEM((2, CH, D), _f32),
                       pltpu.VMEM_SHARED((N, D), _f32),
                       pltpu.SemaphoreType.DMA,
                       pltpu.SemaphoreType.DMA,
                       pltpu.SemaphoreType.DMA,
                       pltpu.SemaphoreType.DMA],
    )
    def sc_scatter(m_hbm, dst_hbm, zero_hbm, agg_hbm, di_v, mv, agg_sh,
                   semm0, semm1, semi0, semi1):
        semm = (semm0, semm1)
        semi = (semi0, semi1)
        cid = lax.axis_index("c")
        sid = lax.axis_index("s")
        wid = sid * NC + cid
        base = wid * EPW
        r0 = sid * RPT
        for b in range(2):
            pltpu.async_copy(dst_hbm.at[pl.ds(base + b * CH, CH)],
                             di_v.at[b], semi[b])
            pltpu.async_copy(m_hbm.at[pl.ds(base + b * CH, CH)],
                             mv.at[b], semm[b])
        pltpu.sync_copy(zero_hbm.at[pl.ds(r0, RPT)], agg_sh.at[pl.ds(r0, RPT)])

        @pl.when(sid == NS - 1)
        def _():
            pltpu.sync_copy(zero_hbm.at[pl.ds(NS * RPT, RTAIL)],
                            agg_sh.at[pl.ds(NS * RPT, RTAIL)])

        plsc.subcore_barrier()

        @pl.loop(0, NCHUNK, step=2)
        def _(j):
            for b in range(2):
                jj = j + b
                pltpu.make_async_copy(dst_hbm.at[pl.ds(0, CH)], di_v.at[b],
                                      semi[b]).wait()
                pltpu.make_async_copy(m_hbm.at[pl.ds(0, CH)], mv.at[b],
                                      semm[b]).wait()
                pltpu.sync_copy(mv.at[b], agg_sh.at[di_v.at[b]], add=True)

                @pl.when(jj + 2 < NCHUNK)
                def _():
                    pltpu.async_copy(
                        dst_hbm.at[pl.ds(base + (jj + 2) * CH, CH)],
                        di_v.at[b], semi[b])
                    pltpu.async_copy(m_hbm.at[pl.ds(base + (jj + 2) * CH, CH)],
                                     mv.at[b], semm[b])

        plsc.subcore_barrier()
        pltpu.sync_copy(agg_sh.at[pl.ds(r0, RPT)],
                        agg_hbm.at[pl.ds(cid * N + r0, RPT)])

        @pl.when(sid == NS - 1)
        def _():
            pltpu.sync_copy(agg_sh.at[pl.ds(NS * RPT, RTAIL)],
                            agg_hbm.at[pl.ds(cid * N + NS * RPT, RTAIL)])

    return sc_gather, sc_scatter


def kernel(x, edge_index, edge_attr, Wf0, bf0, Ws0, bs0, Wu0, bu0,
           Wf1, bf1, Ws1, bs1, Wu1, bu1, Wf2, bf2, Ws2, bs2, Wu2, bu2):
    sc_gather, sc_scatter = _sc_kernels()
    src_h = [edge_index[0, h * EH:(h + 1) * EH].reshape(NW, NCHUNK, CH)
             for h in range(2)]
    dst_h = [edge_index[1, h * EH:(h + 1) * EH].reshape(NW, NCHUNK, CH)
             for h in range(2)]
    dstf_h = [edge_index[1, h * EH:(h + 1) * EH] for h in range(2)]
    ea_h = [edge_attr[h * EH:(h + 1) * EH] for h in range(2)]
    zero = jnp.zeros((N, D), _f32)

    layers = [(Wf0, bf0, Ws0, bs0, Wu0, bu0),
              (Wf1, bf1, Ws1, bs1, Wu1, bu1),
              (Wf2, bf2, Ws2, bs2, Wu2, bu2)]

    agg = None
    u_prev = None
    for li, (Wf, bf, Ws, bs, Wu, bu) in enumerate(layers):
        wfi, wfj, wfe = Wf[:D], Wf[D:2 * D], Wf[2 * D:]
        wsi, wsj, wse = Ws[:D], Ws[D:2 * D], Ws[2 * D:]
        bf2d = bf.reshape(1, D)
        bs2d = bs.reshape(1, D)
        bu2d = bu.reshape(1, D)
        if li == 0:
            p, q, u = _node_call(True)(x, wfi, wsi, wfj, wsj, Wu, bu2d)
        else:
            p, q, u = _node_call(False)(agg[0], agg[0], agg[1], agg[1], u_prev,
                                        wfi, wsi, wfj, wsj, Wu, bu2d)
        agg = []
        for h in range(2):
            fs = sc_gather(p, q, dst_h[h], src_h[h])
            m = _edge_call(fs, ea_h[h], wfe, wse, bf2d, bs2d)
            agg.append(sc_scatter(m, dstf_h[h], zero))
        u_prev = u

    out = _final_call(agg[0], agg[0], agg[1], agg[1], u_prev)
    return out.reshape(1, N, D)


# speedup vs baseline: 7.2266x; 1.0643x over previous
import functools
import jax
import jax.numpy as jnp
from jax import lax
from jax.experimental import pallas as pl
from jax.experimental.pallas import tpu as pltpu
from jax.experimental.pallas import tpu_sc as plsc

N = 10000
E = 640000
EH = E // 2
D = 128
DE = 16
NC = 2
NS = 16
NW = NC * NS
EPW = EH // NW
CH = 80
NCHUNK = EPW // CH
RPT = 624
RTAIL = N - NS * RPT

_f32 = jnp.float32
_i32 = jnp.int32
_u32 = jnp.uint32


def _pack(f_arm, s_arm):
    fb = lax.bitcast_convert_type(f_arm.astype(jnp.bfloat16), jnp.uint16).astype(_u32)
    sb = lax.bitcast_convert_type(s_arm.astype(jnp.bfloat16), jnp.uint16).astype(_u32)
    return lax.bitcast_convert_type(fb | (sb << 16), _i32)


_BR = 1000
_BE = 2000


def _node0_body(x_ref, wfi, wsi, wfj, wsj, wu, bu, p_ref, q_ref, u_ref):
    h = x_ref[...]
    p_ref[...] = _pack(jnp.dot(h, wfi[...], preferred_element_type=_f32),
                       jnp.dot(h, wsi[...], preferred_element_type=_f32))
    q_ref[...] = _pack(jnp.dot(h, wfj[...], preferred_element_type=_f32),
                       jnp.dot(h, wsj[...], preferred_element_type=_f32))
    u_ref[...] = jnp.dot(h, wu[...], preferred_element_type=_f32) + bu[...]


def _node12_body(a0_ref, a1_ref, a2_ref, a3_ref, up_ref,
                 wfi, wsi, wfj, wsj, wu, bu, p_ref, q_ref, u_ref):
    h = (a0_ref[...] + a1_ref[...]) + (a2_ref[...] + a3_ref[...]) + up_ref[...]
    p_ref[...] = _pack(jnp.dot(h, wfi[...], preferred_element_type=_f32),
                       jnp.dot(h, wsi[...], preferred_element_type=_f32))
    q_ref[...] = _pack(jnp.dot(h, wfj[...], preferred_element_type=_f32),
                       jnp.dot(h, wsj[...], preferred_element_type=_f32))
    u_ref[...] = jnp.dot(h, wu[...], preferred_element_type=_f32) + bu[...]


def _edge_body(fs_ref, ea_ref, wfe, wse, bf_ref, bs_ref, m_ref):
    fs = lax.bitcast_convert_type(fs_ref[...], _u32)
    f_arm = lax.bitcast_convert_type(fs << 16, _f32)
    s_arm = lax.bitcast_convert_type(fs & _u32(0xFFFF0000), _f32)
    ea = ea_ref[...]
    f = f_arm + jnp.dot(ea, wfe[...], preferred_element_type=_f32) + bf_ref[...]
    s = s_arm + jnp.dot(ea, wse[...], preferred_element_type=_f32) + bs_ref[...]
    sig = 1.0 / (1.0 + jnp.exp(-f))
    sp = jnp.maximum(s, 0.0) + jnp.log1p(jnp.exp(-jnp.abs(s)))
    m_ref[...] = sig * sp


def _final_body(a0_ref, a1_ref, a2_ref, a3_ref, up_ref, o_ref):
    o_ref[...] = (a0_ref[...] + a1_ref[...]) + (a2_ref[...] + a3_ref[...]) \
        + up_ref[...]


def _node_call(first):
    body = _node0_body if first else _node12_body
    grid = N // _BR
    if first:
        hspec = [pl.BlockSpec((_BR, D), lambda i: (i, 0))]
    else:
        hspec = [pl.BlockSpec((_BR, D), lambda i: (i, 0)),
                 pl.BlockSpec((_BR, D), lambda i: (i + grid, 0)),
                 pl.BlockSpec((_BR, D), lambda i: (i, 0)),
                 pl.BlockSpec((_BR, D), lambda i: (i + grid, 0)),
                 pl.BlockSpec((_BR, D), lambda i: (i, 0))]
    wspecs = [pl.BlockSpec((D, D), lambda i: (0, 0))] * 5 + \
             [pl.BlockSpec((1, D), lambda i: (0, 0))]
    return pl.pallas_call(
        body,
        grid=(grid,),
        in_specs=hspec + wspecs,
        out_specs=[pl.BlockSpec((_BR, D), lambda i: (i, 0)),
                   pl.BlockSpec((_BR, D), lambda i: (i, 0)),
                   pl.BlockSpec((_BR, D), lambda i: (i, 0))],
        out_shape=[jax.ShapeDtypeStruct((N, D), _i32),
                   jax.ShapeDtypeStruct((N, D), _i32),
                   jax.ShapeDtypeStruct((N, D), _f32)],
    )


_edge_call = pl.pallas_call(
    _edge_body,
    grid=(EH // _BE,),
    in_specs=[pl.BlockSpec((_BE, D), lambda i: (i, 0)),
              pl.BlockSpec((_BE, DE), lambda i: (i, 0)),
              pl.BlockSpec((DE, D), lambda i: (0, 0)),
              pl.BlockSpec((DE, D), lambda i: (0, 0)),
              pl.BlockSpec((1, D), lambda i: (0, 0)),
              pl.BlockSpec((1, D), lambda i: (0, 0))],
    out_specs=pl.BlockSpec((_BE, D), lambda i: (i, 0)),
    out_shape=jax.ShapeDtypeStruct((EH, D), _f32),
)

_final_call = pl.pallas_call(
    _final_body,
    grid=(N // _BR,),
    in_specs=[pl.BlockSpec((_BR, D), lambda i: (i, 0)),
              pl.BlockSpec((_BR, D), lambda i: (i + N // _BR, 0)),
              pl.BlockSpec((_BR, D), lambda i: (i, 0)),
              pl.BlockSpec((_BR, D), lambda i: (i + N // _BR, 0)),
              pl.BlockSpec((_BR, D), lambda i: (i, 0))],
    out_specs=pl.BlockSpec((_BR, D), lambda i: (i, 0)),
    out_shape=jax.ShapeDtypeStruct((N, D), _f32),
)


@functools.lru_cache(maxsize=None)
def _sc_kernels():
    mesh = plsc.VectorSubcoreMesh(core_axis_name="c", subcore_axis_name="s",
                                  num_cores=NC, num_subcores=NS)

    @functools.partial(
        pl.kernel,
        out_type=jax.ShapeDtypeStruct((EH, D), _i32),
        mesh=mesh,
        scratch_types=[pltpu.VMEM((EPW,), _i32),
                       pltpu.VMEM((EPW,), _i32),
                       pltpu.VMEM((2, CH, D), _i32),
                       pltpu.VMEM((2, CH, D), _i32),
                       pltpu.VMEM((2, CH, D), _i32),
                       pltpu.SemaphoreType.DMA,
                       pltpu.SemaphoreType.DMA,
                       pltpu.SemaphoreType.DMA,
                       pltpu.SemaphoreType.DMA,
                       pltpu.SemaphoreType.DMA,
                       pltpu.SemaphoreType.DMA],
    )
    def sc_gather(p_hbm, q_hbm, dst_hbm, src_hbm, fs_hbm,
                  di_v, si_v, bd_v, bq_v, wb_v,
                  semp0, semp1, semq0, semq1, semw0, semw1):
        semp = (semp0, semp1)
        semq = (semq0, semq1)
        semw = (semw0, semw1)
        wid = lax.axis_index("s") * NC + lax.axis_index("c")
        base = wid * EPW
        pltpu.sync_copy(dst_hbm.at[pl.ds(base, EPW)], di_v)
        pltpu.sync_copy(src_hbm.at[pl.ds(base, EPW)], si_v)

        def issue(jj, b):
            pltpu.async_copy(p_hbm.at[di_v.at[pl.ds(jj * CH, CH)]],
                             bd_v.at[b], semp[b])
            pltpu.async_copy(q_hbm.at[si_v.at[pl.ds(jj * CH, CH)]],
                             bq_v.at[b], semq[b])

        issue(0, 0)
        issue(1, 1)

        def process(jj, b, tail):
            pltpu.make_async_copy(p_hbm.at[pl.ds(0, CH)], bd_v.at[b],
                                  semp[b]).wait()
            pltpu.make_async_copy(q_hbm.at[pl.ds(0, CH)], bq_v.at[b],
                                  semq[b]).wait()

            def wait_wb():
                pltpu.make_async_copy(
                    wb_v.at[b], fs_hbm.at[pl.ds(0, CH)], semw[b]).wait()

            if tail:
                wait_wb()
            else:
                pl.when(jj >= 2)(wait_wb)

            @pl.loop(0, CH)
            def _(r):
                for g in range(D // 16):
                    sl = pl.ds(g * 16, 16)
                    xd = bd_v[b, r, sl]
                    xq = bq_v[b, r, sl]
                    bc = lax.bitcast_convert_type
                    fsum = bc(xd << 16, _f32) + bc(xq << 16, _f32)
                    ssum = (bc(xd & _i32(-65536), _f32)
                            + bc(xq & _i32(-65536), _f32))
                    rb = _i32(0x8000)
                    lo = ((bc(fsum, _i32) + rb) >> 16) & _i32(0xFFFF)
                    hi = (bc(ssum, _i32) + rb) & _i32(-65536)
                    wb_v[b, r, sl] = lo | hi

            pltpu.async_copy(wb_v.at[b],
                             fs_hbm.at[pl.ds(base + jj * CH, CH)], semw[b])

            if not tail:
                @pl.when(jj + 2 < NCHUNK)
                def _():
                    issue(jj + 2, b)

        @pl.loop(0, NCHUNK - 1, step=2)
        def _(j):
            for b in range(2):
                process(j + b, b, False)

        process(NCHUNK - 1, (NCHUNK - 1) % 2, True)

        for b in range(2):
            pltpu.make_async_copy(wb_v.at[b], fs_hbm.at[pl.ds(0, CH)],
                                  semw[b]).wait()

    @functools.partial(
        pl.kernel,
        out_type=jax.ShapeDtypeStruct((2 * N, D), _f32),
        mesh=mesh,
        scratch_types=[pltpu.VMEM((2, CH), _i32),
                       pltpu.VMEM((2, CH, D), _f32),
                       pltpu.VMEM_SHARED((N, D), _f32),
                       pltpu.SemaphoreType.DMA,
                       pltpu.SemaphoreType.DMA,
                       pltpu.SemaphoreType.DMA,
                       pltpu.SemaphoreType.DMA],
    )
    def sc_scatter(m_hbm, dst_hbm, zero_hbm, agg_hbm, di_v, mv, agg_sh,
                   semm0, semm1, semi0, semi1):
        semm = (semm0, semm1)
        semi = (semi0, semi1)
        cid = lax.axis_index("c")
        sid = lax.axis_index("s")
        wid = sid * NC + cid
        base = wid * EPW
        r0 = sid * RPT
        for b in range(2):
            pltpu.async_copy(dst_hbm.at[pl.ds(base + b * CH, CH)],
                             di_v.at[b], semi[b])
            pltpu.async_copy(m_hbm.at[pl.ds(base + b * CH, CH)],
                             mv.at[b], semm[b])
        pltpu.sync_copy(zero_hbm.at[pl.ds(r0, RPT)], agg_sh.at[pl.ds(r0, RPT)])

        @pl.when(sid == NS - 1)
        def _():
            pltpu.sync_copy(zero_hbm.at[pl.ds(NS * RPT, RTAIL)],
                            agg_sh.at[pl.ds(NS * RPT, RTAIL)])

        plsc.subcore_barrier()

        def sprocess(jj, b, tail):
            pltpu.make_async_copy(dst_hbm.at[pl.ds(0, CH)], di_v.at[b],
                                  semi[b]).wait()
            pltpu.make_async_copy(m_hbm.at[pl.ds(0, CH)], mv.at[b],
                                  semm[b]).wait()
            pltpu.sync_copy(mv.at[b], agg_sh.at[di_v.at[b]], add=True)

            if not tail:
                @pl.when(jj + 2 < NCHUNK)
                def _():
                    pltpu.async_copy(
                        dst_hbm.at[pl.ds(base + (jj + 2) * CH, CH)],
                        di_v.at[b], semi[b])
                    pltpu.async_copy(m_hbm.at[pl.ds(base + (jj + 2) * CH, CH)],
                                     mv.at[b], semm[b])

        @pl.loop(0, NCHUNK - 1, step=2)
        def _(j):
            for b in range(2):
                sprocess(j + b, b, False)

        sprocess(NCHUNK - 1, (NCHUNK - 1) % 2, True)

        plsc.subcore_barrier()
        pltpu.sync_copy(agg_sh.at[pl.ds(r0, RPT)],
                        agg_hbm.at[pl.ds(cid * N + r0, RPT)])

        @pl.when(sid == NS - 1)
        def _():
            pltpu.sync_copy(agg_sh.at[pl.ds(NS * RPT, RTAIL)],
                            agg_hbm.at[pl.ds(cid * N + NS * RPT, RTAIL)])

    return sc_gather, sc_scatter


def kernel(x, edge_index, edge_attr, Wf0, bf0, Ws0, bs0, Wu0, bu0,
           Wf1, bf1, Ws1, bs1, Wu1, bu1, Wf2, bf2, Ws2, bs2, Wu2, bu2):
    sc_gather, sc_scatter = _sc_kernels()
    srcf_h = [edge_index[0, h * EH:(h + 1) * EH] for h in range(2)]
    dstf_h = [edge_index[1, h * EH:(h + 1) * EH] for h in range(2)]
    ea_h = [edge_attr[h * EH:(h + 1) * EH] for h in range(2)]
    zero = jnp.zeros((N, D), _f32)

    layers = [(Wf0, bf0, Ws0, bs0, Wu0, bu0),
              (Wf1, bf1, Ws1, bs1, Wu1, bu1),
              (Wf2, bf2, Ws2, bs2, Wu2, bu2)]

    agg = None
    u_prev = None
    for li, (Wf, bf, Ws, bs, Wu, bu) in enumerate(layers):
        wfi, wfj, wfe = Wf[:D], Wf[D:2 * D], Wf[2 * D:]
        wsi, wsj, wse = Ws[:D], Ws[D:2 * D], Ws[2 * D:]
        bf2d = bf.reshape(1, D)
        bs2d = bs.reshape(1, D)
        bu2d = bu.reshape(1, D)
        if li == 0:
            p, q, u = _node_call(True)(x, wfi, wsi, wfj, wsj, Wu, bu2d)
        else:
            p, q, u = _node_call(False)(agg[0], agg[0], agg[1], agg[1], u_prev,
                                        wfi, wsi, wfj, wsj, Wu, bu2d)
        agg = []
        for h in range(2):
            fs = sc_gather(p, q, dstf_h[h], srcf_h[h])
            m = _edge_call(fs, ea_h[h], wfe, wse, bf2d, bs2d)
            agg.append(sc_scatter(m, dstf_h[h], zero))
        u_prev = u

    out = _final_call(agg[0], agg[0], agg[1], agg[1], u_prev)
    return out.reshape(1, N, D)


# speedup vs baseline: 7.3430x; 1.0161x over previous
import functools
import jax
import jax.numpy as jnp
from jax import lax
from jax.experimental import pallas as pl
from jax.experimental.pallas import tpu as pltpu
from jax.experimental.pallas import tpu_sc as plsc

N = 10000
E = 640000
EH = E // 2
D = 128
DE = 16
NC = 2
NS = 16
NW = NC * NS
EPW = EH // NW
CH = 80
NCHUNK = EPW // CH
RPT = 624
RTAIL = N - NS * RPT

_f32 = jnp.float32
_i32 = jnp.int32
_u32 = jnp.uint32


def _pack(f_arm, s_arm):
    fb = lax.bitcast_convert_type(f_arm.astype(jnp.bfloat16), jnp.uint16).astype(_u32)
    sb = lax.bitcast_convert_type(s_arm.astype(jnp.bfloat16), jnp.uint16).astype(_u32)
    return lax.bitcast_convert_type(fb | (sb << 16), _i32)


_BR = 1000
_BE = 2000


def _node0_body(x_ref, wfi, wsi, wfj, wsj, wu, bu, p_ref, q_ref, u_ref):
    h = x_ref[...]
    p_ref[...] = _pack(jnp.dot(h, wfi[...], preferred_element_type=_f32),
                       jnp.dot(h, wsi[...], preferred_element_type=_f32))
    q_ref[...] = _pack(jnp.dot(h, wfj[...], preferred_element_type=_f32),
                       jnp.dot(h, wsj[...], preferred_element_type=_f32))
    u_ref[...] = jnp.dot(h, wu[...], preferred_element_type=_f32) + bu[...]


def _node12_body(a0_ref, a1_ref, a2_ref, a3_ref, up_ref,
                 wfi, wsi, wfj, wsj, wu, bu, p_ref, q_ref, u_ref):
    h = (a0_ref[...] + a1_ref[...]) + (a2_ref[...] + a3_ref[...]) + up_ref[...]
    p_ref[...] = _pack(jnp.dot(h, wfi[...], preferred_element_type=_f32),
                       jnp.dot(h, wsi[...], preferred_element_type=_f32))
    q_ref[...] = _pack(jnp.dot(h, wfj[...], preferred_element_type=_f32),
                       jnp.dot(h, wsj[...], preferred_element_type=_f32))
    u_ref[...] = jnp.dot(h, wu[...], preferred_element_type=_f32) + bu[...]


def _edge_body(fs_ref, ea_ref, wfe, wse, bf_ref, bs_ref, m_ref):
    fs = lax.bitcast_convert_type(fs_ref[...], _u32)
    f_arm = lax.bitcast_convert_type(fs << 16, _f32)
    s_arm = lax.bitcast_convert_type(fs & _u32(0xFFFF0000), _f32)
    ea = ea_ref[...]
    f = f_arm + jnp.dot(ea, wfe[...], preferred_element_type=_f32) + bf_ref[...]
    s = s_arm + jnp.dot(ea, wse[...], preferred_element_type=_f32) + bs_ref[...]
    sig = 1.0 / (1.0 + jnp.exp(-f))
    sp = jnp.maximum(s, 0.0) + jnp.log1p(jnp.exp(-jnp.abs(s)))
    m_ref[...] = sig * sp


def _final_body(a0_ref, a1_ref, a2_ref, a3_ref, up_ref, o_ref):
    o_ref[...] = (a0_ref[...] + a1_ref[...]) + (a2_ref[...] + a3_ref[...]) \
        + up_ref[...]


def _node_call(first):
    body = _node0_body if first else _node12_body
    grid = N // _BR
    if first:
        hspec = [pl.BlockSpec((_BR, D), lambda i: (i, 0))]
    else:
        hspec = [pl.BlockSpec((_BR, D), lambda i: (i, 0)),
                 pl.BlockSpec((_BR, D), lambda i: (i + grid, 0)),
                 pl.BlockSpec((_BR, D), lambda i: (i, 0)),
                 pl.BlockSpec((_BR, D), lambda i: (i + grid, 0)),
                 pl.BlockSpec((_BR, D), lambda i: (i, 0))]
    wspecs = [pl.BlockSpec((D, D), lambda i: (0, 0))] * 5 + \
             [pl.BlockSpec((1, D), lambda i: (0, 0))]
    return pl.pallas_call(
        body,
        grid=(grid,),
        in_specs=hspec + wspecs,
        out_specs=[pl.BlockSpec((_BR, D), lambda i: (i, 0)),
                   pl.BlockSpec((_BR, D), lambda i: (i, 0)),
                   pl.BlockSpec((_BR, D), lambda i: (i, 0))],
        out_shape=[jax.ShapeDtypeStruct((N, D), _i32),
                   jax.ShapeDtypeStruct((N, D), _i32),
                   jax.ShapeDtypeStruct((N, D), _f32)],
    )


_edge_call = pl.pallas_call(
    _edge_body,
    grid=(EH // _BE,),
    in_specs=[pl.BlockSpec((_BE, D), lambda i: (i, 0)),
              pl.BlockSpec((_BE, DE), lambda i: (i, 0)),
              pl.BlockSpec((DE, D), lambda i: (0, 0)),
              pl.BlockSpec((DE, D), lambda i: (0, 0)),
              pl.BlockSpec((1, D), lambda i: (0, 0)),
              pl.BlockSpec((1, D), lambda i: (0, 0))],
    out_specs=pl.BlockSpec((_BE, D), lambda i: (i, 0)),
    out_shape=jax.ShapeDtypeStruct((EH, D), _f32),
)

_final_call = pl.pallas_call(
    _final_body,
    grid=(N // _BR,),
    in_specs=[pl.BlockSpec((_BR, D), lambda i: (i, 0)),
              pl.BlockSpec((_BR, D), lambda i: (i + N // _BR, 0)),
              pl.BlockSpec((_BR, D), lambda i: (i, 0)),
              pl.BlockSpec((_BR, D), lambda i: (i + N // _BR, 0)),
              pl.BlockSpec((_BR, D), lambda i: (i, 0))],
    out_specs=pl.BlockSpec((_BR, D), lambda i: (i, 0)),
    out_shape=jax.ShapeDtypeStruct((N, D), _f32),
)


@functools.lru_cache(maxsize=None)
def _sc_kernels():
    mesh = plsc.VectorSubcoreMesh(core_axis_name="c", subcore_axis_name="s",
                                  num_cores=NC, num_subcores=NS)

    @functools.partial(
        pl.kernel,
        out_type=jax.ShapeDtypeStruct((EH, D), _i32),
        mesh=mesh,
        scratch_types=[pltpu.VMEM((EPW,), _i32),
                       pltpu.VMEM((EPW,), _i32),
                       pltpu.VMEM((2, CH, D), _i32),
                       pltpu.VMEM((2, CH, D), _i32),
                       pltpu.VMEM((2, CH, D), _i32),
                       pltpu.SemaphoreType.DMA,
                       pltpu.SemaphoreType.DMA,
                       pltpu.SemaphoreType.DMA,
                       pltpu.SemaphoreType.DMA,
                       pltpu.SemaphoreType.DMA,
                       pltpu.SemaphoreType.DMA],
    )
    def sc_gather(p_hbm, q_hbm, dst_hbm, src_hbm, fs_hbm,
                  di_v, si_v, bd_v, bq_v, wb_v,
                  semp0, semp1, semq0, semq1, semw0, semw1):
        semp = (semp0, semp1)
        semq = (semq0, semq1)
        semw = (semw0, semw1)
        wid = lax.axis_index("s") * NC + lax.axis_index("c")
        base = wid * EPW
        pltpu.sync_copy(dst_hbm.at[pl.ds(base, EPW)], di_v)
        pltpu.sync_copy(src_hbm.at[pl.ds(base, EPW)], si_v)

        def issue(jj, b):
            pltpu.async_copy(p_hbm.at[di_v.at[pl.ds(jj * CH, CH)]],
                             bd_v.at[b], semp[b])
            pltpu.async_copy(q_hbm.at[si_v.at[pl.ds(jj * CH, CH)]],
                             bq_v.at[b], semq[b])

        issue(0, 0)
        issue(1, 1)

        def process(jj, b, tail):
            pltpu.make_async_copy(p_hbm.at[pl.ds(0, CH)], bd_v.at[b],
                                  semp[b]).wait()
            pltpu.make_async_copy(q_hbm.at[pl.ds(0, CH)], bq_v.at[b],
                                  semq[b]).wait()

            def wait_wb():
                pltpu.make_async_copy(
                    wb_v.at[b], fs_hbm.at[pl.ds(0, CH)], semw[b]).wait()

            if tail:
                wait_wb()
            else:
                pl.when(jj >= 2)(wait_wb)

            @pl.loop(0, CH)
            def _(r):
                for g in range(D // 16):
                    sl = pl.ds(g * 16, 16)
                    xd = bd_v[b, r, sl]
                    xq = bq_v[b, r, sl]
                    bc = lax.bitcast_convert_type
                    fsum = bc(xd << 16, _f32) + bc(xq << 16, _f32)
                    ssum = (bc(xd & _i32(-65536), _f32)
                            + bc(xq & _i32(-65536), _f32))
                    rb = _i32(0x8000)
                    lo = ((bc(fsum, _i32) + rb) >> 16) & _i32(0xFFFF)
                    hi = (bc(ssum, _i32) + rb) & _i32(-65536)
                    wb_v[b, r, sl] = lo | hi

            pltpu.async_copy(wb_v.at[b],
                             fs_hbm.at[pl.ds(base + jj * CH, CH)], semw[b])

            if not tail:
                @pl.when(jj + 2 < NCHUNK)
                def _():
                    issue(jj + 2, b)

        @pl.loop(0, NCHUNK - 1, step=2)
        def _(j):
            for b in range(2):
                process(j + b, b, False)

        process(NCHUNK - 1, (NCHUNK - 1) % 2, True)

        for b in range(2):
            pltpu.make_async_copy(wb_v.at[b], fs_hbm.at[pl.ds(0, CH)],
                                  semw[b]).wait()

    @functools.partial(
        pl.kernel,
        out_type=jax.ShapeDtypeStruct((2 * N, D), _f32),
        mesh=mesh,
        scratch_types=[pltpu.VMEM((3, CH), _i32),
                       pltpu.VMEM((3, CH, D), _f32),
                       pltpu.VMEM_SHARED((N, D), _f32),
                       pltpu.SemaphoreType.DMA,
                       pltpu.SemaphoreType.DMA,
                       pltpu.SemaphoreType.DMA,
                       pltpu.SemaphoreType.DMA,
                       pltpu.SemaphoreType.DMA,
                       pltpu.SemaphoreType.DMA,
                       pltpu.SemaphoreType.DMA,
                       pltpu.SemaphoreType.DMA,
                       pltpu.SemaphoreType.DMA],
    )
    def sc_scatter(m_hbm, dst_hbm, zero_hbm, agg_hbm, di_v, mv, agg_sh,
                   semm0, semm1, semm2, semi0, semi1, semi2,
                   semsc0, semsc1, semsc2):
        semm = (semm0, semm1, semm2)
        semi = (semi0, semi1, semi2)
        semsc = (semsc0, semsc1, semsc2)
        cid = lax.axis_index("c")
        sid = lax.axis_index("s")
        wid = sid * NC + cid
        base = wid * EPW
        r0 = sid * RPT
        for b in range(2):
            pltpu.async_copy(dst_hbm.at[pl.ds(base + b * CH, CH)],
                             di_v.at[b], semi[b])
            pltpu.async_copy(m_hbm.at[pl.ds(base + b * CH, CH)],
                             mv.at[b], semm[b])
        pltpu.sync_copy(zero_hbm.at[pl.ds(r0, RPT)], agg_sh.at[pl.ds(r0, RPT)])

        @pl.when(sid == NS - 1)
        def _():
            pltpu.sync_copy(zero_hbm.at[pl.ds(NS * RPT, RTAIL)],
                            agg_sh.at[pl.ds(NS * RPT, RTAIL)])

        plsc.subcore_barrier()

        def sprocess(jj, b, first):
            bprev = (b + 2) % 3
            pltpu.make_async_copy(dst_hbm.at[pl.ds(0, CH)], di_v.at[b],
                                  semi[b]).wait()
            pltpu.make_async_copy(m_hbm.at[pl.ds(0, CH)], mv.at[b],
                                  semm[b]).wait()

            def refill():
                pltpu.make_async_copy(m_hbm.at[pl.ds(0, CH)], mv.at[bprev],
                                      semsc[bprev]).wait()

                @pl.when(jj + 2 < NCHUNK)
                def _():
                    pltpu.async_copy(
                        dst_hbm.at[pl.ds(base + (jj + 2) * CH, CH)],
                        di_v.at[bprev], semi[bprev])
                    pltpu.async_copy(m_hbm.at[pl.ds(base + (jj + 2) * CH, CH)],
                                     mv.at[bprev], semm[bprev])

            if first:
                @pl.when(jj + 2 < NCHUNK)
                def _():
                    pltpu.async_copy(
                        dst_hbm.at[pl.ds(base + (jj + 2) * CH, CH)],
                        di_v.at[bprev], semi[bprev])
                    pltpu.async_copy(m_hbm.at[pl.ds(base + (jj + 2) * CH, CH)],
                                     mv.at[bprev], semm[bprev])
            else:
                refill()
            pltpu.async_copy(mv.at[b], agg_sh.at[di_v.at[b]], semsc[b],
                             add=True)

        sprocess(0, 0, True)

        @pl.loop(1, NCHUNK - 1, step=3)
        def _(j):
            for k in range(3):
                jj = j + k
                b = (1 + k) % 3
                sprocess(jj, b, False)

        sprocess(NCHUNK - 1, (NCHUNK - 1) % 3, False)
        pltpu.make_async_copy(m_hbm.at[pl.ds(0, CH)],
                              mv.at[(NCHUNK - 1) % 3],
                              semsc[(NCHUNK - 1) % 3]).wait()

        plsc.subcore_barrier()
        pltpu.sync_copy(agg_sh.at[pl.ds(r0, RPT)],
                        agg_hbm.at[pl.ds(cid * N + r0, RPT)])

        @pl.when(sid == NS - 1)
        def _():
            pltpu.sync_copy(agg_sh.at[pl.ds(NS * RPT, RTAIL)],
                            agg_hbm.at[pl.ds(cid * N + NS * RPT, RTAIL)])

    return sc_gather, sc_scatter


def kernel(x, edge_index, edge_attr, Wf0, bf0, Ws0, bs0, Wu0, bu0,
           Wf1, bf1, Ws1, bs1, Wu1, bu1, Wf2, bf2, Ws2, bs2, Wu2, bu2):
    sc_gather, sc_scatter = _sc_kernels()
    srcf_h = [edge_index[0, h * EH:(h + 1) * EH] for h in range(2)]
    dstf_h = [edge_index[1, h * EH:(h + 1) * EH] for h in range(2)]
    ea_h = [edge_attr[h * EH:(h + 1) * EH] for h in range(2)]
    zero = jnp.zeros((N, D), _f32)

    layers = [(Wf0, bf0, Ws0, bs0, Wu0, bu0),
              (Wf1, bf1, Ws1, bs1, Wu1, bu1),
              (Wf2, bf2, Ws2, bs2, Wu2, bu2)]

    agg = None
    u_prev = None
    for li, (Wf, bf, Ws, bs, Wu, bu) in enumerate(layers):
        wfi, wfj, wfe = Wf[:D], Wf[D:2 * D], Wf[2 * D:]
        wsi, wsj, wse = Ws[:D], Ws[D:2 * D], Ws[2 * D:]
        bf2d = bf.reshape(1, D)
        bs2d = bs.reshape(1, D)
        bu2d = bu.reshape(1, D)
        if li == 0:
            p, q, u = _node_call(True)(x, wfi, wsi, wfj, wsj, Wu, bu2d)
        else:
            p, q, u = _node_call(False)(agg[0], agg[0], agg[1], agg[1], u_prev,
                                        wfi, wsi, wfj, wsj, Wu, bu2d)
        agg = []
        for h in range(2):
            fs = sc_gather(p, q, dstf_h[h], srcf_h[h])
            m = _edge_call(fs, ea_h[h], wfe, wse, bf2d, bs2d)
            agg.append(sc_scatter(m, dstf_h[h], zero))
        u_prev = u

    out = _final_call(agg[0], agg[0], agg[1], agg[1], u_prev)
    return out.reshape(1, N, D)
